# baseline (speedup 1.0000x reference)
import functools

import jax
import jax.numpy as jnp
from jax import lax
from jax.experimental import pallas as pl
from jax.experimental.pallas import tpu as pltpu
from jax.experimental.pallas import tpu_sc as plsc

N = 100000
X = 128
H = 128

_K = 5
_NSLAB = N // _K
_CHUNK = 128
_NC = 2
_NSUB = 16
_GPW = 16
_CPW = 10
_NSP = _GPW * _CPW * _CHUNK
_NBUF = 2


def _gather_body(h_hbm, c_hbm, idx0_hbm, idx1_hbm,
                 h0_out, h1_out, c0_out, c1_out,
                 idx_v, h_v, c_v, gs0, gs1, ws0, ws1):
    wid = lax.axis_index("s") * _NC + lax.axis_index("c")
    gsems = (gs0, gs1)
    wsems = (ws0, ws1)

    def run(idx_hbm, hout_hbm, cout_hbm, lw):
        wbase = lw * _CPW * _CHUNK
        pltpu.sync_copy(idx_hbm.at[pl.ds(wbase, _CPW * _CHUNK)], idx_v)

        def bufs(b):
            return (h_v.at[pl.ds(b * _CHUNK, _CHUNK)],
                    c_v.at[pl.ds(b * _CHUNK, _CHUNK)])

        def issue_gather(j, b):
            iv = idx_v.at[pl.ds(j * _CHUNK, _CHUNK)]
            hbuf, cbuf = bufs(b)
            pltpu.async_copy(h_hbm.at[iv], hbuf, gsems[b])
            pltpu.async_copy(c_hbm.at[iv], cbuf, gsems[b])

        def drain_write(j, b):
            hbuf, cbuf = bufs(b)
            base = wbase + j * _CHUNK
            pltpu.make_async_copy(
                hbuf, hout_hbm.at[pl.ds(base, _CHUNK)], wsems[b]).wait()
            pltpu.make_async_copy(
                cbuf, cout_hbm.at[pl.ds(base, _CHUNK)], wsems[b]).wait()

        issue_gather(0, 0)

        def outer(g, carry):
            for b in range(_NBUF):
                j = g * _NBUF + b
                nb = (b + 1) % _NBUF
                @pl.when(j >= 1)
                def _():
                    drain_write(j - 1, nb)

                @pl.when(j + 1 < _CPW)
                def _():
                    issue_gather(j + 1, nb)

                hbuf, cbuf = bufs(b)
                iv = idx_v.at[pl.ds(j * _CHUNK, _CHUNK)]
                pltpu.make_async_copy(h_hbm.at[iv], hbuf, gsems[b]).wait()
                pltpu.make_async_copy(c_hbm.at[iv], cbuf, gsems[b]).wait()
                base = wbase + j * _CHUNK
                pltpu.async_copy(hbuf, hout_hbm.at[pl.ds(base, _CHUNK)],
                                 wsems[b])
                pltpu.async_copy(cbuf, cout_hbm.at[pl.ds(base, _CHUNK)],
                                 wsems[b])
            return carry

        lax.fori_loop(0, _CPW // _NBUF, outer, 0)
        drain_write(_CPW - 1, (_CPW - 1) % _NBUF)

    @pl.when(wid < _GPW)
    def _():
        run(idx0_hbm, h0_out, c0_out, wid)

    @pl.when(wid >= _GPW)
    def _():
        run(idx1_hbm, h1_out, c1_out, wid - _GPW)


@functools.cache
def _make_sc_gather():
    mail = jax.ShapeDtypeStruct((_NSP, H), jnp.float32)
    return functools.partial(
        pl.kernel,
        mesh=plsc.VectorSubcoreMesh(core_axis_name="c", subcore_axis_name="s"),
        out_type=(mail, mail, mail, mail),
        scratch_types=[
            pltpu.VMEM((_CPW * _CHUNK,), jnp.int32),
            pltpu.VMEM((_NBUF * _CHUNK, H), jnp.float32),
            pltpu.VMEM((_NBUF * _CHUNK, H), jnp.float32),
            pltpu.SemaphoreType.DMA,
            pltpu.SemaphoreType.DMA,
            pltpu.SemaphoreType.DMA,
            pltpu.SemaphoreType.DMA,
        ],
    )(_gather_body)


_BN = 800
_BPS = _NSLAB // _BN


def _dense_body(x_ref, h1_ref, h2_ref, c1_ref, c2_ref, tf_ref,
                wx_ref, wh_ref, wf_ref, biou_ref, bious_ref, bf_ref,
                hacc_ref, cacc_ref, h_out_ref, c_out_ref):
    del hacc_ref, cacc_ref
    x = x_ref[...]
    h1 = h1_ref[...]
    h2 = h2_ref[...]
    wh = wh_ref[...]
    wf = wf_ref[...]
    iou_both = (
        jnp.dot(x, wx_ref[...], preferred_element_type=jnp.float32)
        + jnp.dot(h1, wh[:H], preferred_element_type=jnp.float32)
        + jnp.dot(h2, wh[H:], preferred_element_type=jnp.float32)
    )
    f = jax.nn.sigmoid(
        jnp.dot(h1, wf[:H], preferred_element_type=jnp.float32)
        + jnp.dot(h2, wf[H:], preferred_element_type=jnp.float32)
        + bf_ref[...]
    )
    c1 = c1_ref[...]
    c2 = c2_ref[...]
    c_n = f[:, :H] * c1 + f[:, H:2 * H] * c2
    c_sm = f[:, 2 * H:3 * H] * c1 + f[:, 3 * H:] * c2
    iou_n = iou_both[:, :3 * H] + biou_ref[...]
    iou_sm = iou_both[:, 3 * H:] + bious_ref[...]
    tm = tf_ref[...]
    iou = iou_n + tm * (iou_sm - iou_n)
    c_r = c_n + tm * (c_sm - c_n)
    c_out = jax.nn.sigmoid(iou[:, :H]) * jnp.tanh(iou[:, 2 * H:]) + c_r
    c_out_ref[...] = c_out
    h_out_ref[...] = jax.nn.sigmoid(iou[:, H:2 * H]) * jnp.tanh(c_out)


def _dense_call(s, x, mail, tf, wx, wh, wf, biou, bious, bf, hacc, cacc):
    h0, h1, c0, c1 = mail
    off = s * _BPS
    glob_spec = lambda w: pl.BlockSpec((_BN, w), lambda i: (i + off, 0))
    loc_spec = pl.BlockSpec((_BN, H), lambda i: (i, 0))
    full_spec = lambda a, b: pl.BlockSpec((a, b), lambda i: (0, 0))
    any_spec = pl.BlockSpec(memory_space=pl.ANY)
    return pl.pallas_call(
        _dense_body,
        grid=(_BPS,),
        in_specs=[
            glob_spec(X),
            loc_spec,
            loc_spec,
            loc_spec,
            loc_spec,
            glob_spec(1),
            full_spec(X, 6 * H),
            full_spec(2 * H, 6 * H),
            full_spec(2 * H, 4 * H),
            full_spec(1, 3 * H),
            full_spec(1, 3 * H),
            full_spec(1, 4 * H),
            any_spec,
            any_spec,
        ],
        out_specs=[pl.BlockSpec((_BN, H), lambda i: (i + off, 0)),
                   pl.BlockSpec((_BN, H), lambda i: (i + off, 0))],
        out_shape=[
            jax.ShapeDtypeStruct((N, H), jnp.float32),
            jax.ShapeDtypeStruct((N, H), jnp.float32),
        ],
        input_output_aliases={12: 0, 13: 1},
    )(x, h0, h1, c0, c1, tf, wx, wh, wf, biou, bious, bf, hacc, cacc)


def kernel(x, h_src, c_src, child_idx, t, W_iou, U_iou, b_iou, U_f_w, U_f_b,
           W_iou_s, U_iou_s, b_iou_s, U_f_s_w, U_f_s_b):
    def slab_idx(col):
        pad = jnp.zeros((_K, _NSP - _NSLAB), dtype=jnp.int32)
        return jnp.concatenate(
            [child_idx[:, col].reshape(_K, _NSLAB), pad], axis=1)

    idx0 = slab_idx(0)
    idx1 = slab_idx(1)

    sc_gather = _make_sc_gather()
    mails = [sc_gather(h_src, c_src, idx0[s], idx1[s]) for s in range(_K)]

    wx = jnp.concatenate([W_iou, W_iou_s], axis=1)
    wh = jnp.concatenate(
        [U_iou, jnp.concatenate([U_iou_s, U_iou_s], axis=0)], axis=1)
    z = jnp.zeros((H, H), dtype=jnp.float32)
    ufs_bd = jnp.block([[U_f_s_w, z], [z, U_f_s_w]])
    wf = jnp.concatenate([U_f_w, ufs_bd], axis=1)
    bf = jnp.concatenate([U_f_b, U_f_s_b, U_f_s_b]).reshape(1, 4 * H)
    tf = (t == 1).astype(jnp.float32).reshape(N, 1)

    hacc = None
    cacc = None
    for s in range(_K):
        if s == 0:
            hacc = jnp.zeros((N, H), dtype=jnp.float32)
            cacc = jnp.zeros((N, H), dtype=jnp.float32)
        hacc, cacc = _dense_call(s, x, mails[s], tf, wx, wh, wf,
                                 b_iou, b_iou_s, bf, hacc, cacc)
    return (hacc, cacc)
